# Initial kernel scaffold; baseline (speedup 1.0000x reference)
import jax
import jax.numpy as jnp
from jax.experimental import pallas as pl


def kernel(positions, edge_index, atom_types, W, b):
    raise NotImplementedError("write your pallas kernel here")


# speedup vs baseline: 90.1453x; 90.1453x over previous
import functools

import jax
import jax.numpy as jnp
from jax import lax
from jax.experimental import pallas as pl
from jax.experimental.pallas import tpu as pltpu
from jax.experimental.pallas import tpu_sc as plsc

N = 10000
E = 160000
NT = 3
NMAX = 12
LMAX = 3
Q = 9
CUTOFF = 3.6
CUT_W = 0.3
INV_2SIG2 = 12.5

ROWW = 112
N_PAD = 10016
E_TILE = 10240
E_PAD = 16 * E_TILE
NB = E_TILE // 128
NR = N // 2 * NT
NR_PAD = 15104
STRIPE = NR_PAD // 16
TRASH = NR

SQ3 = 1.7320508075688772
SQ5 = 2.23606797749979
PI = 3.141592653589793


def _vsqrt(x):
    i = lax.bitcast_convert_type(x, jnp.int32)
    i = jnp.int32(0x1FBD1DF5) + lax.shift_right_logical(i, 1)
    y = lax.bitcast_convert_type(i, jnp.float32)
    y = 0.5 * (y + x / y)
    y = 0.5 * (y + x / y)
    y = 0.5 * (y + x / y)
    return y


def _cutoff(r):
    theta = (PI / CUT_W) * (r - (CUTOFF - CUT_W))
    phi = jnp.minimum(theta, PI - theta)
    t = phi * phi
    c = -1.0 / 3628800.0
    c = 1.0 / 40320.0 + t * c
    c = -1.0 / 720.0 + t * c
    c = 1.0 / 24.0 + t * c
    c = -0.5 + t * c
    c = 1.0 + t * c
    cosv = jnp.where(theta > PI * 0.5, -c, c)
    smooth = 0.5 * (1.0 + cosv)
    return jnp.where(r < CUTOFF - CUT_W, 1.0,
                     jnp.where(r < CUTOFF, smooth, 0.0))


_CENTERS = [CUTOFF * n / (NMAX - 1) for n in range(NMAX)]


def _sc_kernel_body(tab_h, src_h, dst_h, out_h,
                    src_v, dst_v, gsrc, gdst, stag, idx_v, cacc):
    core = lax.axis_index("c")
    sub = lax.axis_index("s")
    sc_lo = core * NR

    def _zrow(i, carry):
        for k in range(ROWW // 16):
            stag[i, pl.ds(k * 16, 16)] = jnp.zeros((16,), jnp.float32)
        return carry

    lax.fori_loop(0, 128, _zrow, 0)
    r0 = sub * STRIPE
    for cchunk in range(7):
        pltpu.sync_copy(stag, cacc.at[pl.ds(r0 + cchunk * 128, 128)])
    pltpu.sync_copy(stag.at[pl.ds(0, STRIPE - 896)],
                    cacc.at[pl.ds(r0 + 896, STRIPE - 896)])
    plsc.subcore_barrier()

    tile_base = sub * E_TILE

    def _group(g, carry):
        sl = pl.ds(g * 16, 16)
        evec = g * 16 + lax.iota(jnp.int32, 16)
        dv = dst_v[sl]
        c0 = jnp.full((16,), 0, jnp.int32)
        c1 = jnp.full((16,), 1, jnp.int32)
        c2 = jnp.full((16,), 2, jnp.int32)
        c3 = jnp.full((16,), 3, jnp.int32)
        txf = plsc.load_gather(gsrc, [evec, c3])
        tx = txf.astype(jnp.int32)
        dx = plsc.load_gather(gdst, [evec, c0]) - plsc.load_gather(gsrc, [evec, c0])
        dy = plsc.load_gather(gdst, [evec, c1]) - plsc.load_gather(gsrc, [evec, c1])
        dz = plsc.load_gather(gdst, [evec, c2]) - plsc.load_gather(gsrc, [evec, c2])
        r2 = dx * dx + dy * dy + dz * dz + 1e-12
        r = _vsqrt(r2)
        inv = 1.0 / r
        ux = dx * inv
        uy = dy * inv
        uz = dz * inv
        fc = _cutoff(r)
        y1 = SQ3 * ux
        y2 = SQ3 * uy
        y3 = SQ3 * uz
        y4 = SQ5 * (ux * uy)
        y5 = SQ5 * (uy * uz)
        y6 = SQ5 * (0.5 * (3.0 * (uz * uz) - 1.0))
        y7 = SQ5 * (ux * uz)
        y8 = SQ5 * (0.5 * (ux * ux - uy * uy))

        grow = dv * NT + tx
        lrow = grow - sc_lo
        valid = (lrow >= 0) & (lrow < NR)
        lrow = jnp.where(valid, lrow, TRASH)
        idx_v[sl] = lrow

        for n in range(NMAX):
            d = r - _CENTERS[n]
            a = fc * jnp.exp(-INV_2SIG2 * (d * d))
            vals = (a, a * y1, a * y2, a * y3, a * y4,
                    a * y5, a * y6, a * y7, a * y8)
            for q in range(Q):
                col = jnp.full((16,), n * Q + q, jnp.int32)
                plsc.store_scatter(stag, [evec, col], vals[q])
        return carry

    def _batch(bi, carry):
        base = tile_base + bi * 128
        pltpu.sync_copy(src_h.at[pl.ds(base, 128)], src_v)
        pltpu.sync_copy(dst_h.at[pl.ds(base, 128)], dst_v)
        pltpu.sync_copy(tab_h.at[src_v], gsrc)
        pltpu.sync_copy(tab_h.at[dst_v], gdst)
        lax.fori_loop(0, 8, _group, 0)
        pltpu.sync_copy(stag, cacc.at[idx_v], add=True)
        return carry

    lax.fori_loop(0, NB, _batch, 0)
    plsc.subcore_barrier()

    out_r0 = core * NR_PAD + sub * STRIPE
    pltpu.sync_copy(cacc.at[pl.ds(sub * STRIPE, STRIPE)],
                    out_h.at[pl.ds(out_r0, STRIPE)])


def _make_sc_kernel():
    return functools.partial(
        pl.kernel,
        mesh=plsc.VectorSubcoreMesh(core_axis_name="c", subcore_axis_name="s"),
        out_type=jax.ShapeDtypeStruct((2 * NR_PAD, ROWW), jnp.float32),
        compiler_params=pltpu.CompilerParams(
            needs_layout_passes=False, use_tc_tiling_on_sc=False),
        scratch_types=[
            pltpu.VMEM((128,), jnp.int32),
            pltpu.VMEM((128,), jnp.int32),
            pltpu.VMEM((128, 16), jnp.float32),
            pltpu.VMEM((128, 16), jnp.float32),
            pltpu.VMEM((128, ROWW), jnp.float32),
            pltpu.VMEM((128,), jnp.int32),
            pltpu.VMEM_SHARED((NR_PAD, ROWW), jnp.float32),
        ],
    )(_sc_kernel_body)


def _tc_body(x_ref, b_ref, o_ref):
    x = x_ref[...]
    t = jnp.dot(x, b_ref[...], preferred_element_type=jnp.float32)
    o_ref[...] = jnp.sum(t * x, axis=1, keepdims=True)


def _build_B(W):
    W5 = W.reshape(NT, NMAX, NT, NMAX, LMAX)
    lq = jnp.array([0, 1, 1, 1, 2, 2, 2, 2, 2], dtype=jnp.int32)
    T6 = W5[:, :, :, :, lq]
    B6 = jnp.einsum('anbkq,qr->anqbkr', T6,
                    jnp.eye(Q, dtype=jnp.float32))
    Bm = B6.reshape(NT * NMAX * Q, NT * NMAX * Q)
    return jnp.pad(Bm, ((0, 60), (0, 60)))


def kernel(positions, edge_index, atom_types, W, b):
    src = edge_index[0]
    dst = edge_index[1]
    pad_e = E_PAD - E
    src_p = jnp.concatenate([src, jnp.zeros((pad_e,), jnp.int32)])
    dst_p = jnp.concatenate([dst, jnp.full((pad_e,), N, jnp.int32)])
    tab = jnp.zeros((N_PAD, 16), jnp.float32)
    tab = tab.at[:N, 0:3].set(positions)
    tab = tab.at[:N, 3].set(atom_types.astype(jnp.float32))

    c = _make_sc_kernel()(tab, src_p, dst_p)

    ch = jnp.concatenate([c[0:NR], c[NR_PAD:NR_PAD + NR]], axis=0)
    x = ch.reshape(N, NT, ROWW)[:, :, :NMAX * Q].reshape(N, NT * NMAX * Q)
    x = jnp.pad(x, ((0, 0), (0, 60)))

    Bm = _build_B(W)

    energy = pl.pallas_call(
        _tc_body,
        grid=(50,),
        in_specs=[
            pl.BlockSpec((200, 384), lambda i: (i, 0)),
            pl.BlockSpec((384, 384), lambda i: (0, 0)),
        ],
        out_specs=pl.BlockSpec((200, 1), lambda i: (i, 0)),
        out_shape=jax.ShapeDtypeStruct((N, 1), jnp.float32),
    )(x, Bm)
    return energy + b


# speedup vs baseline: 92.3257x; 1.0242x over previous
import functools

import jax
import jax.numpy as jnp
from jax import lax
from jax.experimental import pallas as pl
from jax.experimental.pallas import tpu as pltpu
from jax.experimental.pallas import tpu_sc as plsc

N = 10000
E = 160000
NT = 3
NMAX = 12
LMAX = 3
Q = 9
CUTOFF = 3.6
CUT_W = 0.3
INV_2SIG2 = 12.5

ROWW = 112
N_PAD = 10016
EB = 64
E_TILE = 10240
E_PAD = 16 * E_TILE
NB = E_TILE // EB
NR = N // 2 * NT
NR_PAD = 15104
STRIPE = NR_PAD // 16
TRASH = NR

SQ3 = 1.7320508075688772
SQ5 = 2.23606797749979
PI = 3.141592653589793


def _vsqrt(x):
    i = lax.bitcast_convert_type(x, jnp.int32)
    i = jnp.int32(0x1FBD1DF5) + lax.shift_right_logical(i, 1)
    y = lax.bitcast_convert_type(i, jnp.float32)
    y = 0.5 * (y + x / y)
    y = 0.5 * (y + x / y)
    y = 0.5 * (y + x / y)
    return y


def _cutoff(r):
    theta = (PI / CUT_W) * (r - (CUTOFF - CUT_W))
    phi = jnp.minimum(theta, PI - theta)
    t = phi * phi
    c = -1.0 / 3628800.0
    c = 1.0 / 40320.0 + t * c
    c = -1.0 / 720.0 + t * c
    c = 1.0 / 24.0 + t * c
    c = -0.5 + t * c
    c = 1.0 + t * c
    cosv = jnp.where(theta > PI * 0.5, -c, c)
    smooth = 0.5 * (1.0 + cosv)
    return jnp.where(r < CUTOFF - CUT_W, 1.0,
                     jnp.where(r < CUTOFF, smooth, 0.0))


_CENTERS = [CUTOFF * n / (NMAX - 1) for n in range(NMAX)]


def _sc_kernel_body(tab_h, src_h, dst_h, out_h,
                    src0, dst0, gs0, gd0, st0, ix0,
                    src1, dst1, gs1, gd1, st1, ix1,
                    sg0, sg1, ss0, ss1, cacc):
    core = lax.axis_index("c")
    sub = lax.axis_index("s")
    sc_lo = core * NR
    sets = ((src0, dst0, gs0, gd0, st0, ix0, sg0, ss0),
            (src1, dst1, gs1, gd1, st1, ix1, sg1, ss1))

    for stag in (st0, st1):
        def _zrow(i, carry, stag=stag):
            for k in range(ROWW // 16):
                stag[i, pl.ds(k * 16, 16)] = jnp.zeros((16,), jnp.float32)
            return carry
        lax.fori_loop(0, EB, _zrow, 0)
    r0 = sub * STRIPE
    for cchunk in range(14):
        pltpu.sync_copy(st0, cacc.at[pl.ds(r0 + cchunk * EB, EB)])
    pltpu.sync_copy(st0.at[pl.ds(0, STRIPE - 896)],
                    cacc.at[pl.ds(r0 + 896, STRIPE - 896)])
    plsc.subcore_barrier()

    tile_base = sub * E_TILE

    def _compute(dst_v, gsrc, gdst, stag, idx_v):
        def _group(g, carry):
            sl = pl.ds(g * 16, 16)
            evec = g * 16 + lax.iota(jnp.int32, 16)
            dv = dst_v[sl]
            c0 = jnp.full((16,), 0, jnp.int32)
            c1 = jnp.full((16,), 1, jnp.int32)
            c2 = jnp.full((16,), 2, jnp.int32)
            c3 = jnp.full((16,), 3, jnp.int32)
            txf = plsc.load_gather(gsrc, [evec, c3])
            tx = txf.astype(jnp.int32)
            dx = (plsc.load_gather(gdst, [evec, c0])
                  - plsc.load_gather(gsrc, [evec, c0]))
            dy = (plsc.load_gather(gdst, [evec, c1])
                  - plsc.load_gather(gsrc, [evec, c1]))
            dz = (plsc.load_gather(gdst, [evec, c2])
                  - plsc.load_gather(gsrc, [evec, c2]))
            r2 = dx * dx + dy * dy + dz * dz + 1e-12
            r = _vsqrt(r2)
            inv = 1.0 / r
            ux = dx * inv
            uy = dy * inv
            uz = dz * inv
            fc = _cutoff(r)
            y1 = SQ3 * ux
            y2 = SQ3 * uy
            y3 = SQ3 * uz
            y4 = SQ5 * (ux * uy)
            y5 = SQ5 * (uy * uz)
            y6 = SQ5 * (0.5 * (3.0 * (uz * uz) - 1.0))
            y7 = SQ5 * (ux * uz)
            y8 = SQ5 * (0.5 * (ux * ux - uy * uy))

            grow = dv * NT + tx
            lrow = grow - sc_lo
            valid = (lrow >= 0) & (lrow < NR)
            lrow = jnp.where(valid, lrow, TRASH)
            idx_v[sl] = lrow

            for n in range(NMAX):
                d = r - _CENTERS[n]
                a = fc * jnp.exp(-INV_2SIG2 * (d * d))
                vals = (a, a * y1, a * y2, a * y3, a * y4,
                        a * y5, a * y6, a * y7, a * y8)
                for q in range(Q):
                    col = jnp.full((16,), n * Q + q, jnp.int32)
                    plsc.store_scatter(stag, [evec, col], vals[q])
            return carry

        lax.fori_loop(0, EB // 16, _group, 0)

    def _ids_load(i, s):
        base = tile_base + i * EB
        pltpu.sync_copy(src_h.at[pl.ds(base, EB)], s[0])
        pltpu.sync_copy(dst_h.at[pl.ds(base, EB)], s[1])

    def _gathers_start(s):
        pltpu.async_copy(tab_h.at[s[0]], s[2], s[6])
        pltpu.async_copy(tab_h.at[s[1]], s[3], s[6])

    def _gathers_wait(s):
        pltpu.make_async_copy(tab_h.at[s[0]], s[2], s[6]).wait()
        pltpu.make_async_copy(tab_h.at[s[1]], s[3], s[6]).wait()

    def _scatter_start(s):
        pltpu.async_copy(s[4], cacc.at[s[5]], s[7], add=True)

    def _scatter_wait(s):
        pltpu.make_async_copy(s[4], cacc.at[s[5]], s[7]).wait()

    _ids_load(0, sets[0])
    _gathers_start(sets[0])

    def _pair(pi, carry):
        for k in (0, 1):
            i = 2 * pi + k
            s = sets[k]
            ns = sets[1 - k]

            @pl.when(i >= 2)
            def _():
                _scatter_wait(s)

            _gathers_wait(s)
            _compute(s[1], s[2], s[3], s[4], s[5])

            @pl.when(i + 1 < NB)
            def _():
                _ids_load(i + 1, ns)
                _gathers_start(ns)

            _scatter_start(s)
        return carry

    lax.fori_loop(0, NB // 2, _pair, 0)
    _scatter_wait(sets[0])
    _scatter_wait(sets[1])
    plsc.subcore_barrier()

    out_r0 = core * NR_PAD + sub * STRIPE
    pltpu.sync_copy(cacc.at[pl.ds(sub * STRIPE, STRIPE)],
                    out_h.at[pl.ds(out_r0, STRIPE)])


def _make_sc_kernel():
    return functools.partial(
        pl.kernel,
        mesh=plsc.VectorSubcoreMesh(core_axis_name="c", subcore_axis_name="s"),
        out_type=jax.ShapeDtypeStruct((2 * NR_PAD, ROWW), jnp.float32),
        compiler_params=pltpu.CompilerParams(
            needs_layout_passes=False, use_tc_tiling_on_sc=False),
        scratch_types=(
            [
                pltpu.VMEM((EB,), jnp.int32),
                pltpu.VMEM((EB,), jnp.int32),
                pltpu.VMEM((EB, 16), jnp.float32),
                pltpu.VMEM((EB, 16), jnp.float32),
                pltpu.VMEM((EB, ROWW), jnp.float32),
                pltpu.VMEM((EB,), jnp.int32),
            ] * 2
            + [
                pltpu.SemaphoreType.DMA,
                pltpu.SemaphoreType.DMA,
                pltpu.SemaphoreType.DMA,
                pltpu.SemaphoreType.DMA,
                pltpu.VMEM_SHARED((NR_PAD, ROWW), jnp.float32),
            ]
        ),
    )(_sc_kernel_body)


def _tc_body(x_ref, b_ref, o_ref):
    x = x_ref[...]
    t = jnp.dot(x, b_ref[...], preferred_element_type=jnp.float32)
    o_ref[...] = jnp.sum(t * x, axis=1, keepdims=True)


def _build_B(W):
    W5 = W.reshape(NT, NMAX, NT, NMAX, LMAX)
    lq = jnp.array([0, 1, 1, 1, 2, 2, 2, 2, 2], dtype=jnp.int32)
    T6 = W5[:, :, :, :, lq]
    B6 = jnp.einsum('anbkq,qr->anqbkr', T6,
                    jnp.eye(Q, dtype=jnp.float32))
    Bm = B6.reshape(NT * NMAX * Q, NT * NMAX * Q)
    return jnp.pad(Bm, ((0, 60), (0, 60)))


def kernel(positions, edge_index, atom_types, W, b):
    src = edge_index[0]
    dst = edge_index[1]
    pad_e = E_PAD - E
    src_p = jnp.concatenate([src, jnp.zeros((pad_e,), jnp.int32)])
    dst_p = jnp.concatenate([dst, jnp.full((pad_e,), N, jnp.int32)])
    tab = jnp.zeros((N_PAD, 16), jnp.float32)
    tab = tab.at[:N, 0:3].set(positions)
    tab = tab.at[:N, 3].set(atom_types.astype(jnp.float32))

    c = _make_sc_kernel()(tab, src_p, dst_p)

    ch = jnp.concatenate([c[0:NR], c[NR_PAD:NR_PAD + NR]], axis=0)
    x = ch.reshape(N, NT, ROWW)[:, :, :NMAX * Q].reshape(N, NT * NMAX * Q)
    x = jnp.pad(x, ((0, 0), (0, 60)))

    Bm = _build_B(W)

    energy = pl.pallas_call(
        _tc_body,
        grid=(50,),
        in_specs=[
            pl.BlockSpec((200, 384), lambda i: (i, 0)),
            pl.BlockSpec((384, 384), lambda i: (0, 0)),
        ],
        out_specs=pl.BlockSpec((200, 1), lambda i: (i, 0)),
        out_shape=jax.ShapeDtypeStruct((N, 1), jnp.float32),
    )(x, Bm)
    return energy + b


# speedup vs baseline: 107.3256x; 1.1625x over previous
import functools

import jax
import jax.numpy as jnp
from jax import lax
from jax.experimental import pallas as pl
from jax.experimental.pallas import tpu as pltpu
from jax.experimental.pallas import tpu_sc as plsc

N = 10000
E = 160000
NT = 3
NMAX = 12
LMAX = 3
Q = 9
CUTOFF = 3.6
CUT_W = 0.3
INV_2SIG2 = 12.5

ROWW = 112
N_PAD = 10016
EB = 64
E_TILE = 10240
E_PAD = 16 * E_TILE
NB = E_TILE // EB
NR = N // 2 * NT
NR_PAD = 15104
STRIPE = NR_PAD // 16
TRASH = NR

SQ3 = 1.7320508075688772
SQ5 = 2.23606797749979
PI = 3.141592653589793


def _vsqrt(x):
    i = lax.bitcast_convert_type(x, jnp.int32)
    i = jnp.int32(0x1FBD1DF5) + lax.shift_right_logical(i, 1)
    y = lax.bitcast_convert_type(i, jnp.float32)
    y = 0.5 * (y + x / y)
    y = 0.5 * (y + x / y)
    y = 0.5 * (y + x / y)
    return y


def _cutoff(r):
    theta = (PI / CUT_W) * (r - (CUTOFF - CUT_W))
    phi = jnp.minimum(theta, PI - theta)
    t = phi * phi
    c = -1.0 / 3628800.0
    c = 1.0 / 40320.0 + t * c
    c = -1.0 / 720.0 + t * c
    c = 1.0 / 24.0 + t * c
    c = -0.5 + t * c
    c = 1.0 + t * c
    cosv = jnp.where(theta > PI * 0.5, -c, c)
    smooth = 0.5 * (1.0 + cosv)
    return jnp.where(r < CUTOFF - CUT_W, 1.0,
                     jnp.where(r < CUTOFF, smooth, 0.0))


_CENTERS = [CUTOFF * n / (NMAX - 1) for n in range(NMAX)]


def _sc_kernel_body(tab_h, src_h, dst_h, out_h,
                    src0, dst0, gs0, gd0, st0, ix0,
                    src1, dst1, gs1, gd1, st1, ix1,
                    sg0, sg1, ss0, ss1, cacc):
    core = lax.axis_index("c")
    sub = lax.axis_index("s")
    sc_lo = core * NR
    sets = ((src0, dst0, gs0, gd0, st0, ix0, sg0, ss0),
            (src1, dst1, gs1, gd1, st1, ix1, sg1, ss1))

    for stag in (st0, st1):
        def _zrow(i, carry, stag=stag):
            for k in range(ROWW // 16):
                stag[i, pl.ds(k * 16, 16)] = jnp.zeros((16,), jnp.float32)
            return carry
        lax.fori_loop(0, EB, _zrow, 0)
    r0 = sub * STRIPE
    for cchunk in range(14):
        pltpu.sync_copy(st0, cacc.at[pl.ds(r0 + cchunk * EB, EB)])
    pltpu.sync_copy(st0.at[pl.ds(0, STRIPE - 896)],
                    cacc.at[pl.ds(r0 + 896, STRIPE - 896)])
    plsc.subcore_barrier()

    tile_base = sub * E_TILE

    def _compute(dst_v, gsrc, gdst, stag, idx_v):
        def _group(g, carry):
            sl = pl.ds(g * 16, 16)
            evec = g * 16 + lax.iota(jnp.int32, 16)
            dv = dst_v[sl]
            c0 = jnp.full((16,), 0, jnp.int32)
            c1 = jnp.full((16,), 1, jnp.int32)
            c2 = jnp.full((16,), 2, jnp.int32)
            c3 = jnp.full((16,), 3, jnp.int32)
            txf = plsc.load_gather(gsrc, [evec, c3])
            tx = txf.astype(jnp.int32)
            dx = (plsc.load_gather(gdst, [evec, c0])
                  - plsc.load_gather(gsrc, [evec, c0]))
            dy = (plsc.load_gather(gdst, [evec, c1])
                  - plsc.load_gather(gsrc, [evec, c1]))
            dz = (plsc.load_gather(gdst, [evec, c2])
                  - plsc.load_gather(gsrc, [evec, c2]))
            r2 = dx * dx + dy * dy + dz * dz + 1e-12
            r = _vsqrt(r2)
            inv = 1.0 / r
            ux = dx * inv
            uy = dy * inv
            uz = dz * inv
            fc = _cutoff(r)
            y1 = SQ3 * ux
            y2 = SQ3 * uy
            y3 = SQ3 * uz
            y4 = SQ5 * (ux * uy)
            y5 = SQ5 * (uy * uz)
            y6 = SQ5 * (0.5 * (3.0 * (uz * uz) - 1.0))
            y7 = SQ5 * (ux * uz)
            y8 = SQ5 * (0.5 * (ux * ux - uy * uy))

            grow = dv * NT + tx
            lrow = grow - sc_lo
            valid = (lrow >= 0) & (lrow < NR)
            lrow = jnp.where(valid, lrow, TRASH)
            idx_v[sl] = lrow

            for n in range(NMAX):
                d = r - _CENTERS[n]
                a = fc * jnp.exp(-INV_2SIG2 * (d * d))
                vals = (a, a * y1, a * y2, a * y3, a * y4,
                        a * y5, a * y6, a * y7, a * y8)
                for q in range(Q):
                    col = jnp.full((16,), n * Q + q, jnp.int32)
                    plsc.store_scatter(stag, [evec, col], vals[q])
            return carry

        lax.fori_loop(0, EB // 16, _group, 0)

    def _ids_load(i, s):
        base = tile_base + i * EB
        pltpu.sync_copy(src_h.at[pl.ds(base, EB)], s[0])
        pltpu.sync_copy(dst_h.at[pl.ds(base, EB)], s[1])

    def _gathers_start(s):
        pltpu.async_copy(tab_h.at[s[0]], s[2], s[6])
        pltpu.async_copy(tab_h.at[s[1]], s[3], s[6])

    def _gathers_wait(s):
        pltpu.make_async_copy(tab_h.at[s[0]], s[2], s[6]).wait()
        pltpu.make_async_copy(tab_h.at[s[1]], s[3], s[6]).wait()

    def _scatter_start(s):
        pltpu.async_copy(s[4], cacc.at[s[5]], s[7], add=True)

    def _scatter_wait(s):
        pltpu.make_async_copy(s[4], cacc.at[s[5]], s[7]).wait()

    _ids_load(0, sets[0])
    _gathers_start(sets[0])

    def _pair(pi, carry):
        for k in (0, 1):
            i = 2 * pi + k
            s = sets[k]
            ns = sets[1 - k]

            @pl.when(i >= 2)
            def _():
                _scatter_wait(s)

            _gathers_wait(s)
            _compute(s[1], s[2], s[3], s[4], s[5])

            @pl.when(i + 1 < NB)
            def _():
                _ids_load(i + 1, ns)
                _gathers_start(ns)

            _scatter_start(s)
        return carry

    lax.fori_loop(0, NB // 2, _pair, 0)
    _scatter_wait(sets[0])
    _scatter_wait(sets[1])
    plsc.subcore_barrier()

    out_r0 = core * NR + sub * STRIPE

    @pl.when(sub < 15)
    def _():
        pltpu.sync_copy(cacc.at[pl.ds(sub * STRIPE, STRIPE)],
                        out_h.at[pl.ds(out_r0, STRIPE)])

    @pl.when(sub == 15)
    def _():
        pltpu.sync_copy(cacc.at[pl.ds(sub * STRIPE, NR - 15 * STRIPE)],
                        out_h.at[pl.ds(out_r0, NR - 15 * STRIPE)])


def _make_sc_kernel():
    return functools.partial(
        pl.kernel,
        mesh=plsc.VectorSubcoreMesh(core_axis_name="c", subcore_axis_name="s"),
        out_type=jax.ShapeDtypeStruct((2 * NR, ROWW), jnp.float32),
        compiler_params=pltpu.CompilerParams(
            needs_layout_passes=False, use_tc_tiling_on_sc=False),
        scratch_types=(
            [
                pltpu.VMEM((EB,), jnp.int32),
                pltpu.VMEM((EB,), jnp.int32),
                pltpu.VMEM((EB, 16), jnp.float32),
                pltpu.VMEM((EB, 16), jnp.float32),
                pltpu.VMEM((EB, ROWW), jnp.float32),
                pltpu.VMEM((EB,), jnp.int32),
            ] * 2
            + [
                pltpu.SemaphoreType.DMA,
                pltpu.SemaphoreType.DMA,
                pltpu.SemaphoreType.DMA,
                pltpu.SemaphoreType.DMA,
                pltpu.VMEM_SHARED((NR_PAD, ROWW), jnp.float32),
            ]
        ),
    )(_sc_kernel_body)


def _tc_body(x_ref, b_ref, o_ref):
    x = x_ref[...]
    t = jnp.dot(x, b_ref[...], preferred_element_type=jnp.float32)
    o_ref[...] = jnp.sum(t * x, axis=1, keepdims=True)


def _build_B(W):
    W5 = W.reshape(NT, NMAX, NT, NMAX, LMAX)
    lq = jnp.array([0, 1, 1, 1, 2, 2, 2, 2, 2], dtype=jnp.int32)
    T6 = W5[:, :, :, :, lq]
    B6 = jnp.einsum('anbkq,qr->anqbkr', T6,
                    jnp.eye(Q, dtype=jnp.float32))
    B4 = B6.reshape(NT, NMAX * Q, NT, NMAX * Q)
    B4 = jnp.pad(B4, ((0, 0), (0, 4), (0, 0), (0, 4)))
    return B4.reshape(NT * ROWW, NT * ROWW)


def kernel(positions, edge_index, atom_types, W, b):
    src = edge_index[0]
    dst = edge_index[1]
    pad_e = E_PAD - E
    src_p = jnp.concatenate([src, jnp.zeros((pad_e,), jnp.int32)])
    dst_p = jnp.concatenate([dst, jnp.full((pad_e,), N, jnp.int32)])
    tab = jnp.zeros((N_PAD, 16), jnp.float32)
    tab = tab.at[:N, 0:3].set(positions)
    tab = tab.at[:N, 3].set(atom_types.astype(jnp.float32))

    c = _make_sc_kernel()(tab, src_p, dst_p)

    x = c.reshape(N, NT * ROWW)
    Bm = _build_B(W)

    energy = pl.pallas_call(
        _tc_body,
        grid=(50,),
        in_specs=[
            pl.BlockSpec((200, NT * ROWW), lambda i: (i, 0)),
            pl.BlockSpec((NT * ROWW, NT * ROWW), lambda i: (0, 0)),
        ],
        out_specs=pl.BlockSpec((200, 1), lambda i: (i, 0)),
        out_shape=jax.ShapeDtypeStruct((N, 1), jnp.float32),
    )(x, Bm)
    return energy + b
